# baseline (speedup 1.0000x reference)
import functools

import jax
import jax.numpy as jnp
from jax import lax
from jax.experimental import pallas as pl
from jax.experimental.pallas import tpu as pltpu
from jax.experimental.pallas import tpu_sc as plsc

K = 1024
D = 64
T = 1024

NC = 2
NS = 16
CHUNK = 1024


def _vq_idx_body(xf_ref, embt_ref, e2_ref, idx_ref):
    xf = xf_ref[...]
    mm = jax.lax.dot_general(
        xf, embt_ref[...], (((1,), (0,)), ((), ())),
        preferred_element_type=jnp.float32)
    x2 = jnp.sum(xf * xf, axis=1, keepdims=True)
    e2 = e2_ref[0:1, :]
    d2 = (x2 + e2) - 2.0 * mm

    dist = jnp.sqrt(jnp.maximum(d2, 0.0))
    m = jnp.min(dist, axis=1, keepdims=True)
    cand = dist == m

    iota = jax.lax.broadcasted_iota(jnp.int32, (T, K), 1)
    idx_ref[...] = jnp.min(jnp.where(cand, iota, K),
                           axis=1, keepdims=True)


def _tc_indices(x, emb):
    n = x.shape[0] * x.shape[2] * x.shape[3]
    xf = jnp.transpose(x, (0, 2, 3, 1)).reshape(-1, D)
    embt = emb.T
    e2 = jnp.sum(emb * emb, axis=1)
    e2b = jnp.broadcast_to(e2[None, :], (8, K))
    idx = pl.pallas_call(
        _vq_idx_body,
        grid=(n // T,),
        in_specs=[
            pl.BlockSpec((T, D), lambda i: (i, 0)),
            pl.BlockSpec((D, K), lambda i: (0, 0)),
            pl.BlockSpec((8, K), lambda i: (0, 0)),
        ],
        out_specs=pl.BlockSpec((T, 1), lambda i: (i, 0)),
        out_shape=jax.ShapeDtypeStruct((n, 1), jnp.int32),
    )(xf, embt, e2b)
    return idx.reshape(n)


def _sc_gather(emb, idx):
    n = idx.shape[0]
    b_per_w = n // (NC * NS)
    mesh = plsc.VectorSubcoreMesh(core_axis_name="c", subcore_axis_name="s")

    @functools.partial(
        pl.kernel, mesh=mesh,
        compiler_params=pltpu.CompilerParams(use_tc_tiling_on_sc=False),
        out_type=jax.ShapeDtypeStruct((n, D), jnp.float32),
        scratch_types=[
            pltpu.VMEM((CHUNK,), jnp.int32),
            pltpu.VMEM((CHUNK, D), jnp.float32),
            pltpu.SemaphoreType.DMA,
        ],
    )
    def gather_k(table_hbm, idx_hbm, out_hbm, idx_v, rows_v, sem):
        wid = lax.axis_index("s") * NC + lax.axis_index("c")
        base = wid * b_per_w
        for c in range(b_per_w // CHUNK):
            off = base + c * CHUNK
            pltpu.sync_copy(idx_hbm.at[pl.ds(off, CHUNK)], idx_v)
            pltpu.async_copy(table_hbm.at[idx_v], rows_v, sem).wait()
            pltpu.sync_copy(rows_v, out_hbm.at[pl.ds(off, CHUNK)])

    return gather_k(emb, idx)


def kernel(x, emb):
    idx = _tc_indices(x, emb)
    return _sc_gather(emb, idx)


# speedup vs baseline: 1.1504x; 1.1504x over previous
import jax
import jax.numpy as jnp
from jax.experimental import pallas as pl

K = 1024
D = 64
T = 2048


def _vq_body(xf_ref, embt_ref, embh_ref, e2_ref, out_ref):
    xf = xf_ref[...]
    mm = jax.lax.dot_general(
        xf, embt_ref[...], (((1,), (0,)), ((), ())),
        preferred_element_type=jnp.float32)
    x2 = jnp.sum(xf * xf, axis=1, keepdims=True)
    e2 = e2_ref[0:1, :]
    d2 = (x2 + e2) - 2.0 * mm

    dist = jnp.sqrt(jnp.maximum(d2, 0.0))
    m = jnp.min(dist, axis=1, keepdims=True)
    cand = dist == m

    iota = jax.lax.broadcasted_iota(jnp.int32, (T, K), 1)
    idx = jnp.min(jnp.where(cand, iota, K), axis=1, keepdims=True)
    onehot = (iota == idx).astype(jnp.bfloat16)
    out_ref[...] = jax.lax.dot_general(
        onehot, embh_ref[...], (((1,), (0,)), ((), ())),
        preferred_element_type=jnp.float32)


def kernel(x, emb):
    n = x.shape[0] * x.shape[2] * x.shape[3]
    xf = jnp.transpose(x, (0, 2, 3, 1)).reshape(-1, D)
    embt = emb.T
    embh = emb.astype(jnp.bfloat16)
    e2 = jnp.sum(emb * emb, axis=1)
    e2b = jnp.broadcast_to(e2[None, :], (8, K))
    return pl.pallas_call(
        _vq_body,
        grid=(n // T,),
        in_specs=[
            pl.BlockSpec((T, D), lambda i: (i, 0)),
            pl.BlockSpec((D, K), lambda i: (0, 0)),
            pl.BlockSpec((K, D), lambda i: (0, 0)),
            pl.BlockSpec((8, K), lambda i: (0, 0)),
        ],
        out_specs=pl.BlockSpec((T, D), lambda i: (i, 0)),
        out_shape=jax.ShapeDtypeStruct((n, D), jnp.float32),
    )(xf, embt, embh, e2b)


# speedup vs baseline: 1.1756x; 1.0219x over previous
import jax
import jax.numpy as jnp
from jax.experimental import pallas as pl

K = 1024
D = 64
T = 4096


def _vq_body(xf_ref, embt_ref, embh_ref, e2_ref, out_ref):
    xf = xf_ref[...]
    mm = jax.lax.dot_general(
        xf, embt_ref[...], (((1,), (0,)), ((), ())),
        preferred_element_type=jnp.float32)
    x2 = jnp.sum(xf * xf, axis=1, keepdims=True)
    e2 = e2_ref[0:1, :]
    d2 = (x2 + e2) - 2.0 * mm

    dist = jnp.sqrt(jnp.maximum(d2, 0.0))
    m = jnp.min(dist, axis=1, keepdims=True)
    cand = dist == m

    iota = jax.lax.broadcasted_iota(jnp.int32, (T, K), 1)
    idx = jnp.min(jnp.where(cand, iota, K), axis=1, keepdims=True)
    onehot = (iota == idx).astype(jnp.bfloat16)
    out_ref[...] = jax.lax.dot_general(
        onehot, embh_ref[...], (((1,), (0,)), ((), ())),
        preferred_element_type=jnp.float32)


def kernel(x, emb):
    n = x.shape[0] * x.shape[2] * x.shape[3]
    xf = jnp.transpose(x, (0, 2, 3, 1)).reshape(-1, D)
    embt = emb.T
    embh = emb.astype(jnp.bfloat16)
    e2 = jnp.sum(emb * emb, axis=1)
    e2b = jnp.broadcast_to(e2[None, :], (8, K))
    return pl.pallas_call(
        _vq_body,
        grid=(n // T,),
        in_specs=[
            pl.BlockSpec((T, D), lambda i: (i, 0)),
            pl.BlockSpec((D, K), lambda i: (0, 0)),
            pl.BlockSpec((K, D), lambda i: (0, 0)),
            pl.BlockSpec((8, K), lambda i: (0, 0)),
        ],
        out_specs=pl.BlockSpec((T, D), lambda i: (i, 0)),
        out_shape=jax.ShapeDtypeStruct((n, D), jnp.float32),
    )(xf, embt, embh, e2b)
